# baseline (speedup 1.0000x reference)
import functools

import jax
import jax.numpy as jnp
from jax import lax
from jax.experimental import pallas as pl
from jax.experimental.pallas import tpu as pltpu
from jax.experimental.pallas import tpu_sc as plsc

N = 10000
D = 128
E = 320000
NC = 2
NS = 16
NW = NC * NS
C = 125
NR = 3
NQ = 4
NCH = 80
EPW = NCH * C
EPAD = NW * EPW
AGG_ROWS = 10112
ZROWS = AGG_ROWS // NS
RB = 2000


def _sc_scatter(m, src_idx, dst_idx, zeros):
    mesh = plsc.VectorSubcoreMesh(core_axis_name="c", subcore_axis_name="s")

    @functools.partial(
        pl.kernel,
        out_type=jax.ShapeDtypeStruct((NC, AGG_ROWS, D), jnp.float32),
        mesh=mesh,
        scratch_types=[
            pltpu.VMEM((NQ, C), jnp.int32),
            pltpu.VMEM((NQ, C), jnp.int32),
            pltpu.VMEM((NR, C, D), jnp.float32),
            pltpu.VMEM_SHARED((AGG_ROWS, D), jnp.float32),
            pltpu.SemaphoreType.DMA,
            pltpu.SemaphoreType.DMA,
            pltpu.SemaphoreType.DMA,
        ],
    )
    def k(m_hbm, src_hbm, dst_hbm, z_hbm, out_hbm, srcq, dstq, rows_v, agg_sh,
          gsem, isem, ssem):
        cid = lax.axis_index("c")
        sid = lax.axis_index("s")
        pltpu.sync_copy(z_hbm.at[pl.ds(sid * ZROWS, ZROWS)],
                        agg_sh.at[pl.ds(sid * ZROWS, ZROWS)])
        plsc.subcore_barrier()

        def idx_start(g, slot):
            pltpu.async_copy(src_hbm.at[cid, sid, g], srcq.at[slot], isem)
            pltpu.async_copy(dst_hbm.at[cid, sid, g], dstq.at[slot], isem)

        def idx_wait(g, slot):
            pltpu.make_async_copy(src_hbm.at[cid, sid, g], srcq.at[slot],
                                  isem).wait()
            pltpu.make_async_copy(dst_hbm.at[cid, sid, g], dstq.at[slot],
                                  isem).wait()

        def gather_start(qslot, rslot):
            pltpu.async_copy(m_hbm.at[srcq.at[qslot]], rows_v.at[rslot], gsem)

        def gather_wait(rslot):
            pltpu.make_async_copy(m_hbm.at[srcq.at[0]], rows_v.at[rslot],
                                  gsem).wait()

        def scat_start(qslot, rslot):
            pltpu.async_copy(rows_v.at[rslot], agg_sh.at[dstq.at[qslot]],
                             ssem, add=True)

        def scat_wait(rslot):
            pltpu.make_async_copy(rows_v.at[rslot], agg_sh.at[dstq.at[0]],
                                  ssem).wait()

        for q in range(NQ):
            idx_start(q, q)
        for b in range(2):
            idx_wait(b, b)
            gather_start(b, b)

        def body(g, carry):
            b = g % NR
            qb = g % NQ
            gather_wait(b)
            scat_start(qb, b)

            @pl.when(g >= 1)
            def _():
                scat_wait((g - 1) % NR)

                @pl.when(g + 3 < NCH)
                def _():
                    idx_start(g + 3, (g - 1) % NQ)

            @pl.when(g + 2 < NCH)
            def _():
                idx_wait(g + 2, (g + 2) % NQ)
                gather_start((g + 2) % NQ, (g + 2) % NR)

            return carry

        lax.fori_loop(0, NCH, body, 0)
        scat_wait((NCH - 1) % NR)
        plsc.subcore_barrier()
        pltpu.sync_copy(agg_sh.at[pl.ds(sid * ZROWS, ZROWS)],
                        out_hbm.at[cid, pl.ds(sid * ZROWS, ZROWS)])

    return k(m, src_idx, dst_idx, zeros)


def _matmul(h, w):
    def body(h_ref, w_ref, o_ref):
        o_ref[...] = jnp.dot(h_ref[...], w_ref[...],
                             preferred_element_type=jnp.float32)

    return pl.pallas_call(
        body,
        grid=(N // RB,),
        in_specs=[pl.BlockSpec((RB, D), lambda i: (i, 0)),
                  pl.BlockSpec((D, D), lambda i: (0, 0))],
        out_specs=pl.BlockSpec((RB, D), lambda i: (i, 0)),
        out_shape=jax.ShapeDtypeStruct((N, D), jnp.float32),
    )(h, w)


def _gru(p, h, wihT, whhT, bih, bhh, w_next):
    fuse = w_next is not None

    def body(p_ref, h_ref, wih_ref, whh_ref, bih_ref, bhh_ref, *rest):
        if fuse:
            wn_ref, h_out_ref, m_out_ref = rest
        else:
            (h_out_ref,) = rest
        agg = p_ref[0] + p_ref[1]
        h_blk = h_ref[...]
        gi = jnp.dot(agg, wih_ref[...], preferred_element_type=jnp.float32) + bih_ref[...]
        gh = jnp.dot(h_blk, whh_ref[...], preferred_element_type=jnp.float32) + bhh_ref[...]
        r = jax.nn.sigmoid(gi[:, :D] + gh[:, :D])
        z = jax.nn.sigmoid(gi[:, D:2 * D] + gh[:, D:2 * D])
        n = jnp.tanh(gi[:, 2 * D:] + r * gh[:, 2 * D:])
        hn = (1.0 - z) * n + z * h_blk
        h_out_ref[...] = hn
        if fuse:
            m_out_ref[...] = jnp.dot(hn, wn_ref[...],
                                     preferred_element_type=jnp.float32)

    in_specs = [
        pl.BlockSpec((NC, RB, D), lambda i: (0, i, 0)),
        pl.BlockSpec((RB, D), lambda i: (i, 0)),
        pl.BlockSpec((D, 3 * D), lambda i: (0, 0)),
        pl.BlockSpec((D, 3 * D), lambda i: (0, 0)),
        pl.BlockSpec((1, 3 * D), lambda i: (0, 0)),
        pl.BlockSpec((1, 3 * D), lambda i: (0, 0)),
    ]
    out_shape = jax.ShapeDtypeStruct((N, D), jnp.float32)
    if fuse:
        in_specs.append(pl.BlockSpec((D, D), lambda i: (0, 0)))
        out_shape = (out_shape, jax.ShapeDtypeStruct((N, D), jnp.float32))
    out_specs = pl.BlockSpec((RB, D), lambda i: (i, 0))
    if fuse:
        out_specs = (out_specs, pl.BlockSpec((RB, D), lambda i: (i, 0)))

    args = (p, h, wihT, whhT, bih, bhh) + ((w_next,) if fuse else ())
    return pl.pallas_call(
        body,
        grid=(N // RB,),
        in_specs=in_specs,
        out_specs=out_specs,
        out_shape=out_shape,
    )(*args)


def kernel(x, edge_index, weight, W_ih, W_hh, b_ih, b_hh):
    num_layers = weight.shape[0]
    src = edge_index[0]
    dst = edge_index[1]
    padw = EPW - E // NW
    wcol = jnp.arange(NW, dtype=jnp.int32)[:, None]
    jcol = jnp.arange(padw, dtype=jnp.int32)[None, :]
    pad_src = ((wcol * padw + jcol) * 997) % N
    pad_dst = N + (wcol * padw + jcol) % (AGG_ROWS - N)
    src_p = jnp.concatenate([src.reshape(NW, E // NW), pad_src], axis=1)
    dst_p = jnp.concatenate([dst.reshape(NW, E // NW), pad_dst], axis=1)
    src_p = src_p.reshape(NC, NS, NCH, C)
    dst_p = dst_p.reshape(NC, NS, NCH, C)
    zeros = jnp.zeros((AGG_ROWS, D), jnp.float32)
    wihT = W_ih.T
    whhT = W_hh.T
    bih = b_ih.reshape(1, 3 * D)
    bhh = b_hh.reshape(1, 3 * D)

    h = x
    m = _matmul(h, weight[0])
    for i in range(num_layers):
        p = _sc_scatter(m, src_p, dst_p, zeros)
        if i < num_layers - 1:
            h, m = _gru(p, h, wihT, whhT, bih, bhh, weight[i + 1])
        else:
            h = _gru(p, h, wihT, whhT, bih, bhh, None)
    return h


# speedup vs baseline: 1.0116x; 1.0116x over previous
import functools

import jax
import jax.numpy as jnp
from jax import lax
from jax.experimental import pallas as pl
from jax.experimental.pallas import tpu as pltpu
from jax.experimental.pallas import tpu_sc as plsc

N = 10000
D = 128
E = 320000
NC = 2
NS = 16
NW = NC * NS
C = 125
NR = 3
NQ = 4
NCH = 80
EPW = NCH * C
EPAD = NW * EPW
AGG_ROWS = 10112
ZROWS = AGG_ROWS // NS
RB = 2000


def _sc_scatter(m, src_idx, dst_idx, zeros):
    mesh = plsc.VectorSubcoreMesh(core_axis_name="c", subcore_axis_name="s")

    @functools.partial(
        pl.kernel,
        out_type=jax.ShapeDtypeStruct((NC, AGG_ROWS, D), jnp.float32),
        mesh=mesh,
        scratch_types=[
            pltpu.VMEM((NQ, C), jnp.int32),
            pltpu.VMEM((NQ, C), jnp.int32),
            pltpu.VMEM((NR, C, D), jnp.float32),
            pltpu.VMEM_SHARED((AGG_ROWS, D), jnp.float32),
            pltpu.SemaphoreType.DMA,
            pltpu.SemaphoreType.DMA,
            pltpu.SemaphoreType.DMA,
        ],
    )
    def k(m_hbm, src_hbm, dst_hbm, z_hbm, out_hbm, srcq, dstq, rows_v, agg_sh,
          gsem, isem, ssem):
        cid = lax.axis_index("c")
        sid = lax.axis_index("s")

        def idx_start(g, slot):
            pltpu.async_copy(src_hbm.at[cid, sid, g], srcq.at[slot], isem)
            pltpu.async_copy(dst_hbm.at[cid, sid, g], dstq.at[slot], isem)

        def idx_wait(g, slot):
            pltpu.make_async_copy(src_hbm.at[cid, sid, g], srcq.at[slot],
                                  isem).wait()
            pltpu.make_async_copy(dst_hbm.at[cid, sid, g], dstq.at[slot],
                                  isem).wait()

        def gather_start(qslot, rslot):
            pltpu.async_copy(m_hbm.at[srcq.at[qslot]], rows_v.at[rslot], gsem)

        def gather_wait(rslot):
            pltpu.make_async_copy(m_hbm.at[srcq.at[0]], rows_v.at[rslot],
                                  gsem).wait()

        def scat_start(qslot, rslot):
            pltpu.async_copy(rows_v.at[rslot], agg_sh.at[dstq.at[qslot]],
                             ssem, add=True)

        def scat_wait(rslot):
            pltpu.make_async_copy(rows_v.at[rslot], agg_sh.at[dstq.at[0]],
                                  ssem).wait()

        for q in range(NQ):
            idx_start(q, q)
        pltpu.sync_copy(z_hbm.at[pl.ds(sid * ZROWS, ZROWS)],
                        agg_sh.at[pl.ds(sid * ZROWS, ZROWS)])
        for b in range(2):
            idx_wait(b, b)
            gather_start(b, b)
        plsc.subcore_barrier()

        def body(g, carry):
            b = g % NR
            qb = g % NQ
            gather_wait(b)
            scat_start(qb, b)

            @pl.when(g >= 1)
            def _():
                scat_wait((g - 1) % NR)

                @pl.when(g + 3 < NCH)
                def _():
                    idx_start(g + 3, (g - 1) % NQ)

            @pl.when(g + 2 < NCH)
            def _():
                idx_wait(g + 2, (g + 2) % NQ)
                gather_start((g + 2) % NQ, (g + 2) % NR)

            return carry

        lax.fori_loop(0, NCH, body, 0)
        scat_wait((NCH - 1) % NR)
        plsc.subcore_barrier()
        pltpu.sync_copy(agg_sh.at[pl.ds(sid * ZROWS, ZROWS)],
                        out_hbm.at[cid, pl.ds(sid * ZROWS, ZROWS)])

    return k(m, src_idx, dst_idx, zeros)


def _matmul(h, w):
    def body(h_ref, w_ref, o_ref):
        o_ref[...] = jnp.dot(h_ref[...], w_ref[...],
                             preferred_element_type=jnp.float32)

    return pl.pallas_call(
        body,
        grid=(N // RB,),
        in_specs=[pl.BlockSpec((RB, D), lambda i: (i, 0)),
                  pl.BlockSpec((D, D), lambda i: (0, 0))],
        out_specs=pl.BlockSpec((RB, D), lambda i: (i, 0)),
        out_shape=jax.ShapeDtypeStruct((N, D), jnp.float32),
    )(h, w)


def _gru(p, h, wihT, whhT, bih, bhh, w_next):
    fuse = w_next is not None

    def body(p_ref, h_ref, wih_ref, whh_ref, bih_ref, bhh_ref, *rest):
        if fuse:
            wn_ref, h_out_ref, m_out_ref = rest
        else:
            (h_out_ref,) = rest
        agg = p_ref[0] + p_ref[1]
        h_blk = h_ref[...]
        gi = jnp.dot(agg, wih_ref[...], preferred_element_type=jnp.float32) + bih_ref[...]
        gh = jnp.dot(h_blk, whh_ref[...], preferred_element_type=jnp.float32) + bhh_ref[...]
        r = jax.nn.sigmoid(gi[:, :D] + gh[:, :D])
        z = jax.nn.sigmoid(gi[:, D:2 * D] + gh[:, D:2 * D])
        n = jnp.tanh(gi[:, 2 * D:] + r * gh[:, 2 * D:])
        hn = (1.0 - z) * n + z * h_blk
        h_out_ref[...] = hn
        if fuse:
            m_out_ref[...] = jnp.dot(hn, wn_ref[...],
                                     preferred_element_type=jnp.float32)

    in_specs = [
        pl.BlockSpec((NC, RB, D), lambda i: (0, i, 0)),
        pl.BlockSpec((RB, D), lambda i: (i, 0)),
        pl.BlockSpec((D, 3 * D), lambda i: (0, 0)),
        pl.BlockSpec((D, 3 * D), lambda i: (0, 0)),
        pl.BlockSpec((1, 3 * D), lambda i: (0, 0)),
        pl.BlockSpec((1, 3 * D), lambda i: (0, 0)),
    ]
    out_shape = jax.ShapeDtypeStruct((N, D), jnp.float32)
    if fuse:
        in_specs.append(pl.BlockSpec((D, D), lambda i: (0, 0)))
        out_shape = (out_shape, jax.ShapeDtypeStruct((N, D), jnp.float32))
    out_specs = pl.BlockSpec((RB, D), lambda i: (i, 0))
    if fuse:
        out_specs = (out_specs, pl.BlockSpec((RB, D), lambda i: (i, 0)))

    args = (p, h, wihT, whhT, bih, bhh) + ((w_next,) if fuse else ())
    return pl.pallas_call(
        body,
        grid=(N // RB,),
        in_specs=in_specs,
        out_specs=out_specs,
        out_shape=out_shape,
    )(*args)


def kernel(x, edge_index, weight, W_ih, W_hh, b_ih, b_hh):
    num_layers = weight.shape[0]
    src = edge_index[0]
    dst = edge_index[1]
    padw = EPW - E // NW
    wcol = jnp.arange(NW, dtype=jnp.int32)[:, None]
    jcol = jnp.arange(padw, dtype=jnp.int32)[None, :]
    pad_src = ((wcol * padw + jcol) * 997) % N
    pad_dst = N + (wcol * padw + jcol) % (AGG_ROWS - N)
    src_p = jnp.concatenate([src.reshape(NW, E // NW), pad_src], axis=1)
    dst_p = jnp.concatenate([dst.reshape(NW, E // NW), pad_dst], axis=1)
    src_p = src_p.reshape(NC, NS, NCH, C)
    dst_p = dst_p.reshape(NC, NS, NCH, C)
    zeros = jnp.zeros((AGG_ROWS, D), jnp.float32)
    wihT = W_ih.T
    whhT = W_hh.T
    bih = b_ih.reshape(1, 3 * D)
    bhh = b_hh.reshape(1, 3 * D)

    h = x
    m = _matmul(h, weight[0])
    for i in range(num_layers):
        p = _sc_scatter(m, src_p, dst_p, zeros)
        if i < num_layers - 1:
            h, m = _gru(p, h, wihT, whhT, bih, bhh, weight[i + 1])
        else:
            h = _gru(p, h, wihT, whhT, bih, bhh, None)
    return h
